# baseline (speedup 1.0000x reference)
import functools

import jax
import jax.numpy as jnp
from jax import lax
from jax.experimental import pallas as pl
from jax.experimental.pallas import tpu as pltpu

N_NODES = 50000
N_PAD = 50176
BLK = 512


def _layer_tc_body(apply_act, agg_ref, bprev_ref, w_ref, a_s_ref, a_d_ref,
                   h_ref, asn_ref, adn_ref):
    xin = agg_ref[...]
    if apply_act:
        xin = jnp.maximum(xin + bprev_ref[...], 0.0)
    h = jnp.dot(xin, w_ref[...], preferred_element_type=jnp.float32)
    h_ref[...] = h
    asn_ref[...] = jnp.dot(h, a_s_ref[...], preferred_element_type=jnp.float32)
    adn_ref[...] = jnp.dot(h, a_d_ref[...], preferred_element_type=jnp.float32)


def _layer_transform(agg, bprev, W, a_s, a_d, apply_act):
    din = W.shape[0]
    dout = W.shape[1]
    grid = N_PAD // BLK
    body = functools.partial(_layer_tc_body, apply_act)
    h, asn, adn = pl.pallas_call(
        body,
        grid=(grid,),
        in_specs=[
            pl.BlockSpec((BLK, din), lambda i: (i, 0)),
            pl.BlockSpec((1, din), lambda i: (0, 0)),
            pl.BlockSpec((din, dout), lambda i: (0, 0)),
            pl.BlockSpec((dout, 1), lambda i: (0, 0)),
            pl.BlockSpec((dout, 1), lambda i: (0, 0)),
        ],
        out_specs=[
            pl.BlockSpec((BLK, dout), lambda i: (i, 0)),
            pl.BlockSpec((BLK, 1), lambda i: (i, 0)),
            pl.BlockSpec((BLK, 1), lambda i: (i, 0)),
        ],
        out_shape=[
            jax.ShapeDtypeStruct((N_PAD, dout), jnp.float32),
            jax.ShapeDtypeStruct((N_PAD, 1), jnp.float32),
            jax.ShapeDtypeStruct((N_PAD, 1), jnp.float32),
        ],
    )(agg, bprev.reshape(1, din), W, a_s.reshape(dout, 1),
      a_d.reshape(dout, 1))
    return h, asn[:, 0], adn[:, 0]


def _head_body(g_ref, w1_ref, b1_ref, w2_ref, b2_ref, l1w_ref, l1b_ref,
               l2w_ref, l2b_ref, l3w_ref, l3b_ref, out_ref):
    g = g_ref[...]
    g = jnp.maximum(jnp.dot(g, w1_ref[...], preferred_element_type=jnp.float32)
                    + b1_ref[...], 0.0)
    g = jnp.dot(g, w2_ref[...], preferred_element_type=jnp.float32) + b2_ref[...]
    g = jax.nn.sigmoid(g)
    t = jnp.maximum(jnp.dot(g, l1w_ref[...], preferred_element_type=jnp.float32)
                    + l1b_ref[...], 0.0)
    t = jnp.maximum(jnp.dot(t, l2w_ref[...], preferred_element_type=jnp.float32)
                    + l2b_ref[...], 0.0)
    t = jnp.maximum(jnp.dot(t, l3w_ref[...], preferred_element_type=jnp.float32)
                    + l3b_ref[...], 0.0)
    out_ref[...] = jax.nn.sigmoid(t)


def _mlp_head(g, fc_g1_w, fc_g1_b, fc_g2_w, fc_g2_b, l1_w, l1_b, l2_w, l2_b,
              l3_w, l3_b):
    G = g.shape[0]
    return pl.pallas_call(
        _head_body,
        out_shape=jax.ShapeDtypeStruct((G, 1), jnp.float32),
    )(g, fc_g1_w, fc_g1_b.reshape(1, -1), fc_g2_w, fc_g2_b.reshape(1, -1),
      l1_w, l1_b.reshape(1, -1), l2_w, l2_b.reshape(1, -1), l3_w,
      l3_b.reshape(1, -1))


def _edge_aggregate(h, asn, adn, src, dst):
    e = asn[src] + adn[dst]
    e = jnp.where(e >= 0.0, e, 0.2 * e)
    m = jax.ops.segment_max(e, dst, num_segments=N_NODES)
    ex = jnp.exp(e - m[dst])
    denom = jax.ops.segment_sum(ex, dst, num_segments=N_NODES)
    alpha = ex / (denom[dst] + 1e-16)
    agg = jax.ops.segment_sum(alpha[:, None] * h[src], dst,
                              num_segments=N_NODES)
    return jnp.pad(agg, ((0, N_PAD - N_NODES), (0, 0)))


def kernel(x, edge_index, fp, batch, W1, a_s1, a_d1, b1, W2, a_s2, a_d2, b2,
           W3, a_s3, a_d3, b3, fc_g1_w, fc_g1_b, fc_g2_w, fc_g2_b, l1_w, l1_b,
           l2_w, l2_b, l3_w, l3_b):
    src, dst = edge_index[0], edge_index[1]
    G = fp.shape[0]

    xp = jnp.pad(x, ((0, N_PAD - N_NODES), (0, 0)))
    zeros32 = jnp.zeros((32,), jnp.float32)

    h1, as1, ad1 = _layer_transform(xp, zeros32, W1, a_s1, a_d1, False)
    agg1 = _edge_aggregate(h1, as1, ad1, src, dst)

    h2, as2, ad2 = _layer_transform(agg1, b1, W2, a_s2, a_d2, True)
    agg2 = _edge_aggregate(h2, as2, ad2, src, dst)

    h3, as3, ad3 = _layer_transform(agg2, b2, W3, a_s3, a_d3, True)
    agg3 = _edge_aggregate(h3, as3, ad3, src, dst)

    hfin = jnp.maximum(agg3[:N_NODES] + b3, 0.0)
    g = jax.ops.segment_max(hfin, batch, num_segments=G)
    return _mlp_head(g, fc_g1_w, fc_g1_b, fc_g2_w, fc_g2_b, l1_w, l1_b,
                     l2_w, l2_b, l3_w, l3_b)
